# Initial kernel scaffold; baseline (speedup 1.0000x reference)
import jax
import jax.numpy as jnp
from jax.experimental import pallas as pl


def kernel(bias, emb_dest, emb_src, feature_src, enc_W1, enc_b1, enc_W2, enc_b2, dec_W1, dec_b1, dec_W2, dec_b2, att_W, att_W2):
    raise NotImplementedError("write your pallas kernel here")


# speedup vs baseline: 1.4640x; 1.4640x over previous
import functools

import jax
import jax.numpy as jnp
from jax.experimental import pallas as pl
from jax.experimental.pallas import tpu as pltpu

N_DEST = 4096
N_SRC = 4096
FEAT = 1024
TFEAT = 256
EMB = 64
HID = 128
HEADS = 4

D_BLK = 256
S_BLK = 1024
N_I = N_DEST // D_BLK
N_J = N_SRC // S_BLK

def _dot(a, b, dims):
    return jax.lax.dot_general(
        a.astype(jnp.bfloat16), b.astype(jnp.bfloat16), dims,
        preferred_element_type=jnp.float32,
    )


def _mm(a, b):
    return _dot(a, b, (((1,), (0,)), ((), ())))


def _encoder_kernel(x_ref, w1_ref, b1_ref, w2_ref, b2_ref, out_ref):
    h = jnp.maximum(_mm(x_ref[...], w1_ref[...]) + b1_ref[...], 0.0)
    out_ref[...] = _mm(h, w2_ref[...]) + b2_ref[...]


def _encoder(x, w1, b1, w2, b2):
    blk = 1024
    return pl.pallas_call(
        _encoder_kernel,
        grid=(N_SRC // blk,),
        in_specs=[
            pl.BlockSpec((blk, FEAT), lambda i: (i, 0)),
            pl.BlockSpec((FEAT, 2 * TFEAT), lambda i: (0, 0)),
            pl.BlockSpec((1, 2 * TFEAT), lambda i: (0, 0)),
            pl.BlockSpec((2 * TFEAT, TFEAT), lambda i: (0, 0)),
            pl.BlockSpec((1, TFEAT), lambda i: (0, 0)),
        ],
        out_specs=pl.BlockSpec((blk, TFEAT), lambda i: (i, 0)),
        out_shape=jax.ShapeDtypeStruct((N_SRC, TFEAT), jnp.float32),
    )(x, w1, b1, w2, b2)


def _decoder_kernel(tf_ref, w1_ref, b1_ref, w2_ref, b2_ref, out_ref):
    h = jnp.maximum(_mm(tf_ref[...], w1_ref[...]) + b1_ref[...], 0.0)
    out_ref[...] = _mm(h, w2_ref[...]) + b2_ref[...]


def _decoder(tf, w1, b1, w2, b2):
    blk = 1024
    return pl.pallas_call(
        _decoder_kernel,
        grid=(N_SRC // blk,),
        in_specs=[
            pl.BlockSpec((blk, TFEAT), lambda i: (i, 0)),
            pl.BlockSpec((TFEAT, 2 * TFEAT), lambda i: (0, 0)),
            pl.BlockSpec((1, 2 * TFEAT), lambda i: (0, 0)),
            pl.BlockSpec((2 * TFEAT, FEAT), lambda i: (0, 0)),
            pl.BlockSpec((1, FEAT), lambda i: (0, 0)),
        ],
        out_specs=pl.BlockSpec((blk, FEAT), lambda i: (i, 0)),
        out_shape=jax.ShapeDtypeStruct((N_SRC, FEAT), jnp.float32),
    )(tf, w1, b1, w2, b2)


def _qk_kernel(ed_ref, es_ref, w_ref, w2_ref, q_ref, kt_ref):
    for h in range(HEADS):
        h2 = _mm(ed_ref[...], w_ref[h])
        q_ref[h] = _mm(h2, w2_ref[h])
        kt_ref[h] = _dot(w_ref[h], es_ref[...], (((0,), (1,)), ((), ())))


def _qk(emb_dest, emb_src, att_w, att_w2):
    return pl.pallas_call(
        _qk_kernel,
        grid=(1,),
        in_specs=[
            pl.BlockSpec((N_DEST, EMB), lambda i: (0, 0)),
            pl.BlockSpec((N_SRC, EMB), lambda i: (0, 0)),
            pl.BlockSpec((HEADS, EMB, HID), lambda i: (0, 0, 0)),
            pl.BlockSpec((HEADS, HID, HID), lambda i: (0, 0, 0)),
        ],
        out_specs=[
            pl.BlockSpec((HEADS, N_DEST, HID), lambda i: (0, 0, 0)),
            pl.BlockSpec((HEADS, HID, N_SRC), lambda i: (0, 0, 0)),
        ],
        out_shape=[
            jax.ShapeDtypeStruct((HEADS, N_DEST, HID), jnp.float32),
            jax.ShapeDtypeStruct((HEADS, HID, N_SRC), jnp.float32),
        ],
    )(emb_dest, emb_src, att_w, att_w2)


def _attn_kernel(q_ref, kt_ref, bias_ref, tf_ref, out_ref,
                 acc_ref, m_ref, l_ref):
    j = pl.program_id(1)

    @pl.when(j == 0)
    def _init():
        acc_ref[...] = jnp.zeros_like(acc_ref)
        m_ref[...] = jnp.full_like(m_ref, -jnp.inf)
        l_ref[...] = jnp.zeros_like(l_ref)

    bias_pos = bias_ref[...] > 0.0
    tf_blk = tf_ref[...]

    for h in range(HEADS):
        s = _mm(q_ref[h], kt_ref[h])
        s = jnp.where(s >= 0.0, s, 0.2 * s)
        s = jnp.where(bias_pos, s, -9e15)
        m_prev = m_ref[h]
        l_prev = l_ref[h]
        m_cur = jnp.max(s, axis=1, keepdims=True)
        m_new = jnp.maximum(m_prev, m_cur)
        alpha = jnp.exp(m_prev - m_new)
        p = jnp.exp(s - m_new)
        l_ref[h] = alpha * l_prev + jnp.sum(p, axis=1, keepdims=True)
        m_ref[h] = m_new
        acc_ref[h] = acc_ref[h] * alpha + _mm(p, tf_blk)

    @pl.when(j == N_J - 1)
    def _finalize():
        out = jnp.zeros((D_BLK, TFEAT), jnp.float32)
        for h in range(HEADS):
            hp = acc_ref[h] / l_ref[h]
            out = out + jnp.where(hp > 0.0, hp, jnp.exp(hp) - 1.0)
        out_ref[...] = out * (1.0 / HEADS)


def _attention(q, kt, bias, tf):
    return pl.pallas_call(
        _attn_kernel,
        grid=(N_I, N_J),
        in_specs=[
            pl.BlockSpec((HEADS, D_BLK, HID), lambda i, j: (0, i, 0)),
            pl.BlockSpec((HEADS, HID, S_BLK), lambda i, j: (0, 0, j)),
            pl.BlockSpec((D_BLK, S_BLK), lambda i, j: (i, j)),
            pl.BlockSpec((S_BLK, TFEAT), lambda i, j: (j, 0)),
        ],
        out_specs=pl.BlockSpec((D_BLK, TFEAT), lambda i, j: (i, 0)),
        out_shape=jax.ShapeDtypeStruct((N_DEST, TFEAT), jnp.float32),
        scratch_shapes=[
            pltpu.VMEM((HEADS, D_BLK, TFEAT), jnp.float32),
            pltpu.VMEM((HEADS, D_BLK, 1), jnp.float32),
            pltpu.VMEM((HEADS, D_BLK, 1), jnp.float32),
        ],
        compiler_params=pltpu.CompilerParams(
            dimension_semantics=("arbitrary", "arbitrary"),
        ),
    )(q, kt, bias, tf)


def kernel(bias, emb_dest, emb_src, feature_src, enc_W1, enc_b1, enc_W2,
           enc_b2, dec_W1, dec_b1, dec_W2, dec_b2, att_W, att_W2):
    tf = _encoder(feature_src, enc_W1, enc_b1.reshape(1, -1), enc_W2,
                  enc_b2.reshape(1, -1))
    q, kt = _qk(emb_dest, emb_src, att_W, att_W2)
    feature_src_re = _attention(q, kt, bias, tf)
    feature_hat = _decoder(tf, dec_W1, dec_b1.reshape(1, -1), dec_W2,
                           dec_b2.reshape(1, -1))
    return (feature_src_re, feature_hat, tf)


# speedup vs baseline: 2.0357x; 1.3905x over previous
import jax
import jax.numpy as jnp
from jax.experimental import pallas as pl
from jax.experimental.pallas import tpu as pltpu

N_DEST = 4096
N_SRC = 4096
FEAT = 1024
TFEAT = 256
EMB = 64
HID = 128
HEADS = 4

D_BLK = 256
E_BLK = 1024


def _dot(a, b, dims):
    return jax.lax.dot_general(
        a.astype(jnp.bfloat16), b.astype(jnp.bfloat16), dims,
        preferred_element_type=jnp.float32,
    )


def _mm(a, b):
    return _dot(a, b, (((1,), (0,)), ((), ())))


def _enc_qk_kernel(x_ref, w1_ref, b1_ref, w2_ref, b2_ref,
                   ed_ref, es_ref, aw_ref, aw2_ref,
                   tf_ref, q_ref, kt_ref):
    @pl.when(pl.program_id(0) == 0)
    def _qk():
        for h in range(HEADS):
            h2 = _mm(ed_ref[...], aw_ref[h])
            q_ref[h] = _mm(h2, aw2_ref[h])
            kt_ref[h] = _dot(aw_ref[h], es_ref[...], (((0,), (1,)), ((), ())))

    h = jnp.maximum(_mm(x_ref[...], w1_ref[...]) + b1_ref[...], 0.0)
    tf_ref[...] = _mm(h, w2_ref[...]) + b2_ref[...]


def _enc_qk(x, w1, b1, w2, b2, emb_dest, emb_src, att_w, att_w2):
    return pl.pallas_call(
        _enc_qk_kernel,
        grid=(N_SRC // E_BLK,),
        in_specs=[
            pl.BlockSpec((E_BLK, FEAT), lambda i: (i, 0)),
            pl.BlockSpec((FEAT, 2 * TFEAT), lambda i: (0, 0)),
            pl.BlockSpec((1, 2 * TFEAT), lambda i: (0, 0)),
            pl.BlockSpec((2 * TFEAT, TFEAT), lambda i: (0, 0)),
            pl.BlockSpec((1, TFEAT), lambda i: (0, 0)),
            pl.BlockSpec((N_DEST, EMB), lambda i: (0, 0)),
            pl.BlockSpec((N_SRC, EMB), lambda i: (0, 0)),
            pl.BlockSpec((HEADS, EMB, HID), lambda i: (0, 0, 0)),
            pl.BlockSpec((HEADS, HID, HID), lambda i: (0, 0, 0)),
        ],
        out_specs=[
            pl.BlockSpec((E_BLK, TFEAT), lambda i: (i, 0)),
            pl.BlockSpec((HEADS, N_DEST, HID), lambda i: (0, 0, 0)),
            pl.BlockSpec((HEADS, HID, N_SRC), lambda i: (0, 0, 0)),
        ],
        out_shape=[
            jax.ShapeDtypeStruct((N_SRC, TFEAT), jnp.float32),
            jax.ShapeDtypeStruct((HEADS, N_DEST, HID), jnp.float32),
            jax.ShapeDtypeStruct((HEADS, HID, N_SRC), jnp.float32),
        ],
    )(x, w1, b1, w2, b2, emb_dest, emb_src, att_w, att_w2)


def _attn_dec_kernel(q_ref, kt_ref, bias_ref, tf_ref,
                     dw1_ref, db1_ref, dw2_ref, db2_ref,
                     out_ref, fh_ref):
    bias_pos = bias_ref[...] > 0.0
    tf_all = tf_ref[...]

    out = jnp.zeros((D_BLK, TFEAT), jnp.float32)
    for h in range(HEADS):
        s = _mm(q_ref[h], kt_ref[h])
        s = jnp.maximum(s, 0.2 * s)
        s = jnp.where(bias_pos, s, -9e15)
        m = jnp.max(s, axis=1, keepdims=True)
        p = jnp.exp(s - m)
        l = jnp.sum(p, axis=1, keepdims=True)
        hp = _mm(p, tf_all) * (1.0 / l)
        out = out + jnp.where(hp > 0.0, hp, jnp.exp(hp) - 1.0)
    out_ref[...] = out * (1.0 / HEADS)

    r0 = pl.multiple_of(pl.program_id(0) * D_BLK, D_BLK)
    tfb = tf_ref[pl.ds(r0, D_BLK), :]
    hid = jnp.maximum(_mm(tfb, dw1_ref[...]) + db1_ref[...], 0.0)
    fh_ref[...] = _mm(hid, dw2_ref[...]) + db2_ref[...]


def _attn_dec(q, kt, bias, tf, dw1, db1, dw2, db2):
    return pl.pallas_call(
        _attn_dec_kernel,
        grid=(N_DEST // D_BLK,),
        in_specs=[
            pl.BlockSpec((HEADS, D_BLK, HID), lambda i: (0, i, 0)),
            pl.BlockSpec((HEADS, HID, N_SRC), lambda i: (0, 0, 0)),
            pl.BlockSpec((D_BLK, N_SRC), lambda i: (i, 0)),
            pl.BlockSpec((N_SRC, TFEAT), lambda i: (0, 0)),
            pl.BlockSpec((TFEAT, 2 * TFEAT), lambda i: (0, 0)),
            pl.BlockSpec((1, 2 * TFEAT), lambda i: (0, 0)),
            pl.BlockSpec((2 * TFEAT, FEAT), lambda i: (0, 0)),
            pl.BlockSpec((1, FEAT), lambda i: (0, 0)),
        ],
        out_specs=[
            pl.BlockSpec((D_BLK, TFEAT), lambda i: (i, 0)),
            pl.BlockSpec((D_BLK, FEAT), lambda i: (i, 0)),
        ],
        out_shape=[
            jax.ShapeDtypeStruct((N_DEST, TFEAT), jnp.float32),
            jax.ShapeDtypeStruct((N_SRC, FEAT), jnp.float32),
        ],
    )(q, kt, bias, tf, dw1, db1, dw2, db2)


def kernel(bias, emb_dest, emb_src, feature_src, enc_W1, enc_b1, enc_W2,
           enc_b2, dec_W1, dec_b1, dec_W2, dec_b2, att_W, att_W2):
    tf, q, kt = _enc_qk(feature_src, enc_W1, enc_b1.reshape(1, -1),
                        enc_W2, enc_b2.reshape(1, -1),
                        emb_dest, emb_src, att_W, att_W2)
    feature_src_re, feature_hat = _attn_dec(
        q, kt, bias, tf, dec_W1, dec_b1.reshape(1, -1),
        dec_W2, dec_b2.reshape(1, -1))
    return (feature_src_re, feature_hat, tf)
